# baseline (speedup 1.0000x reference)
import functools

import jax
import jax.numpy as jnp
from jax import lax
from jax.experimental import pallas as pl
from jax.experimental.pallas import tpu as pltpu
from jax.experimental.pallas import tpu_sc as plsc

N = 10000
NUM_HE = 10000
E = 320000
XD = 128
HD = 128
GD = 128
PD = HD + GD

NC = 2
NS = 16
K = 128
NBJ = 2
TCH = 2560
C0W = 80
C1W = TCH // NS - C0W
EP = TCH * K
NP = 10112
DUMP = N
RPW = NP // NS
R = 1000


def _tc1_body(f_ref, t_ref, wphi_ref, bphi_ref, whg_ref, phi_ref, xw_ref):
    phi = jnp.maximum(
        jnp.dot(f_ref[...], wphi_ref[...], preferred_element_type=jnp.float32)
        + bphi_ref[...], 0.0)
    phi_ref[...] = phi
    xw_ref[...] = jnp.dot(t_ref[...] * phi, whg_ref[...],
                          preferred_element_type=jnp.float32)


_tc1 = pl.pallas_call(
    _tc1_body,
    grid=(N // R,),
    in_specs=[
        pl.BlockSpec((R, XD), lambda i: (i, 0)),
        pl.BlockSpec((R, 1), lambda i: (i, 0)),
        pl.BlockSpec((XD, HD), lambda i: (0, 0)),
        pl.BlockSpec((1, HD), lambda i: (0, 0)),
        pl.BlockSpec((HD, GD), lambda i: (0, 0)),
    ],
    out_specs=[
        pl.BlockSpec((R, HD), lambda i: (i, 0)),
        pl.BlockSpec((R, GD), lambda i: (i, 0)),
    ],
    out_shape=[
        jax.ShapeDtypeStruct((N, HD), jnp.float32),
        jax.ShapeDtypeStruct((N, GD), jnp.float32),
    ],
)


def _tc2_body(pe_ref, pb_ref, ef_ref):
    s = pe_ref[0] + pe_ref[1]
    b = pb_ref[0] + pb_ref[1]
    binv = jnp.where(b > 0, 1.0 / jnp.maximum(b, 1.0), 0.0)
    ef_ref[...] = binv * s


_tc2 = pl.pallas_call(
    _tc2_body,
    grid=(N // R,),
    in_specs=[
        pl.BlockSpec((NC, R, GD), lambda i: (0, i, 0)),
        pl.BlockSpec((NC, R, 1), lambda i: (0, i, 0)),
    ],
    out_specs=pl.BlockSpec((R, GD), lambda i: (i, 0)),
    out_shape=jax.ShapeDtypeStruct((N, GD), jnp.float32),
)


def _tc3_body(pn_ref, pd_ref, phi_ref, bias_ref,
              w00a_ref, w00b_ref, b00_ref,
              w10a_ref, w10b_ref, b10_ref,
              w01_ref, b01_ref, w11_ref, b11_ref,
              y1_ref, y0_ref):
    q = pn_ref[0] + pn_ref[1]
    d = pd_ref[0] + pd_ref[1]
    dinv = jnp.where(d > 0, 1.0 / jnp.maximum(d, 1.0), 0.0)
    rep = dinv * q + bias_ref[...]
    phi = phi_ref[...]
    y00 = jnp.maximum(
        jnp.dot(phi, w00a_ref[...], preferred_element_type=jnp.float32)
        + jnp.dot(rep, w00b_ref[...], preferred_element_type=jnp.float32)
        + b00_ref[...], 0.0)
    y10 = jnp.maximum(
        jnp.dot(phi, w10a_ref[...], preferred_element_type=jnp.float32)
        + jnp.dot(rep, w10b_ref[...], preferred_element_type=jnp.float32)
        + b10_ref[...], 0.0)
    y0_ref[...] = jnp.dot(y00, w01_ref[...],
                          preferred_element_type=jnp.float32) + b01_ref[...]
    y1_ref[...] = jnp.dot(y10, w11_ref[...],
                          preferred_element_type=jnp.float32) + b11_ref[...]


_tc3 = pl.pallas_call(
    _tc3_body,
    grid=(N // R,),
    in_specs=[
        pl.BlockSpec((NC, R, GD), lambda i: (0, i, 0)),
        pl.BlockSpec((NC, R, 1), lambda i: (0, i, 0)),
        pl.BlockSpec((R, HD), lambda i: (i, 0)),
        pl.BlockSpec((1, GD), lambda i: (0, 0)),
        pl.BlockSpec((HD, PD), lambda i: (0, 0)),
        pl.BlockSpec((GD, PD), lambda i: (0, 0)),
        pl.BlockSpec((1, PD), lambda i: (0, 0)),
        pl.BlockSpec((HD, PD), lambda i: (0, 0)),
        pl.BlockSpec((GD, PD), lambda i: (0, 0)),
        pl.BlockSpec((1, PD), lambda i: (0, 0)),
        pl.BlockSpec((PD, 1), lambda i: (0, 0)),
        pl.BlockSpec((1, 1), lambda i: (0, 0)),
        pl.BlockSpec((PD, 1), lambda i: (0, 0)),
        pl.BlockSpec((1, 1), lambda i: (0, 0)),
    ],
    out_specs=[
        pl.BlockSpec((R, 1), lambda i: (i, 0)),
        pl.BlockSpec((R, 1), lambda i: (i, 0)),
    ],
    out_shape=[
        jax.ShapeDtypeStruct((N, 1), jnp.float32),
        jax.ShapeDtypeStruct((N, 1), jnp.float32),
    ],
)


def _build_sc_pass(mesh, with_counts):
    out_type = [jax.ShapeDtypeStruct((NC, NP, GD), jnp.float32)]
    scratch = [
        pltpu.VMEM_SHARED((NP, GD), jnp.float32),
        pltpu.VMEM((2, NBJ, K), jnp.int32),
        pltpu.VMEM((2, NBJ, K), jnp.int32),
        pltpu.VMEM((NBJ, K, GD), jnp.float32),
    ]
    if with_counts:
        out_type += [jax.ShapeDtypeStruct((NC, NP), jnp.float32),
                     jax.ShapeDtypeStruct((NC, NP), jnp.float32)]
        scratch += [pltpu.VMEM_SHARED((NP,), jnp.float32),
                    pltpu.VMEM_SHARED((NP,), jnp.float32),
                    pltpu.VMEM((K,), jnp.float32)]
    scratch += [pltpu.SemaphoreType.DMA] * (2 * NBJ + 1)

    @functools.partial(pl.kernel, mesh=mesh, out_type=out_type,
                       scratch_types=scratch)
    def sc_pass(*refs):
        if with_counts:
            (table, gidx0, sidx0, gidx1, sidx1, zrows, zcnt, ones_hbm,
             acc_out, pb_out, pd_out,
             acc, gring, sring, rows, bcnt, dcnt, ones, *sems) = refs
        else:
            (table, gidx0, sidx0, gidx1, sidx1, zrows,
             acc_out, acc, gring, sring, rows, *sems) = refs
        gsem = sems[:NBJ]
        ssem = sems[NBJ:2 * NBJ]
        isem = sems[2 * NBJ]
        cid = lax.axis_index("c")
        sid = lax.axis_index("s")

        pltpu.sync_copy(zrows, acc.at[pl.ds(sid * RPW, RPW)])
        if with_counts:
            @pl.when(sid == 0)
            def _zero_counts():
                pltpu.sync_copy(zcnt, bcnt)
                pltpu.sync_copy(zcnt, dcnt)
            pltpu.sync_copy(ones_hbm, ones)

        def run_loop(gidx, sidx, nburst):
            pltpu.sync_copy(gidx.at[sid, pl.ds(0, NBJ)], gring.at[0])
            pltpu.sync_copy(sidx.at[sid, pl.ds(0, NBJ)], sring.at[0])

            def outer(t, carry):
                p = lax.rem(t, 2)
                q = 1 - p
                tn = jnp.minimum(t + 1, nburst - 1)
                dg = [pltpu.async_copy(table.at[gring.at[p, j]], rows.at[j],
                                       gsem[j])
                      for j in range(NBJ)]
                di = [pltpu.async_copy(gidx.at[sid, pl.ds(tn * NBJ, NBJ)],
                                       gring.at[q], isem),
                      pltpu.async_copy(sidx.at[sid, pl.ds(tn * NBJ, NBJ)],
                                       sring.at[q], isem)]
                ds = []
                for j in range(NBJ):
                    dg[j].wait()
                    ds.append(pltpu.async_copy(rows.at[j],
                                               acc.at[sring.at[p, j]],
                                               ssem[j], add=True))
                    if with_counts:
                        ds.append(pltpu.async_copy(ones,
                                                   bcnt.at[sring.at[p, j]],
                                                   ssem[j], add=True))
                        ds.append(pltpu.async_copy(ones,
                                                   dcnt.at[gring.at[p, j]],
                                                   ssem[j], add=True))
                for d in ds + di:
                    d.wait()
                return carry

            lax.fori_loop(0, nburst, outer, 0)

        @pl.when(cid == 0)
        def _core0():
            run_loop(gidx0, sidx0, C0W // NBJ)

        @pl.when(cid == 1)
        def _core1():
            run_loop(gidx1, sidx1, C1W // NBJ)

        plsc.subcore_barrier()
        pltpu.sync_copy(acc.at[pl.ds(sid * RPW, RPW)],
                        acc_out.at[cid, pl.ds(sid * RPW, RPW)])
        if with_counts:
            @pl.when(sid == 0)
            def _flush_counts():
                pltpu.sync_copy(bcnt, pb_out.at[cid])
                pltpu.sync_copy(dcnt, pd_out.at[cid])

    return sc_pass


@functools.lru_cache(maxsize=1)
def _sc_kernels():
    mesh = plsc.VectorSubcoreMesh(core_axis_name="c", subcore_axis_name="s")
    return _build_sc_pass(mesh, True), _build_sc_pass(mesh, False)


def kernel(features, treatments, hyperedge_index, W_phi, b_phi, W_hg, bias_hg,
           W_t00, b_t00, W_t10, b_t10, W_t01, b_t01, W_t11, b_t11):
    phi_x, xw = _tc1(features, treatments.reshape(N, 1), W_phi,
                     b_phi.reshape(1, HD), W_hg)

    node_idx = hyperedge_index[0]
    edge_idx = hyperedge_index[1]
    pad = EP - E
    pad0 = jnp.zeros((pad,), jnp.int32)
    padd = jnp.full((pad,), DUMP, jnp.int32)
    split = NS * C0W * K

    def _split(idx, padval):
        flat = jnp.concatenate([idx, padval])
        return (flat[:split].reshape(NS, C0W, K),
                flat[split:].reshape(NS, C1W, K))

    n_g0, n_g1 = _split(node_idx, pad0)
    n_s0, n_s1 = _split(node_idx, padd)
    e_g0, e_g1 = _split(edge_idx, pad0)
    e_s0, e_s1 = _split(edge_idx, padd)

    zrows = jnp.zeros((RPW, GD), jnp.float32)
    zcnt = jnp.zeros((NP,), jnp.float32)
    onearr = jnp.ones((K,), jnp.float32)

    sc_pass_a, sc_pass_b = _sc_kernels()
    pe, pb, pd = sc_pass_a(xw, n_g0, e_s0, n_g1, e_s1, zrows, zcnt, onearr)
    ef = _tc2(pe, pb.reshape(NC, NP, 1))
    [pn] = sc_pass_b(ef, e_g0, n_s0, e_g1, n_s1, zrows)
    y1, y0 = _tc3(pn, pd.reshape(NC, NP, 1), phi_x, bias_hg.reshape(1, GD),
                  W_t00[:HD], W_t00[HD:], b_t00.reshape(1, PD),
                  W_t10[:HD], W_t10[HD:], b_t10.reshape(1, PD),
                  W_t01, b_t01.reshape(1, 1),
                  W_t11, b_t11.reshape(1, 1))
    return y1.reshape(-1), y0.reshape(-1), phi_x


# speedup vs baseline: 1.2778x; 1.2778x over previous
import functools

import jax
import jax.numpy as jnp
from jax import lax
from jax.experimental import pallas as pl
from jax.experimental.pallas import tpu as pltpu
from jax.experimental.pallas import tpu_sc as plsc

N = 10000
NUM_HE = 10000
E = 320000
XD = 128
HD = 128
GD = 128
PD = HD + GD

NC = 2
NS = 16
K = 128
NBJ = 2
GI = 80
NBURST = GI // NBJ
EP = NC * NS * GI * K
NP = 10112
DUMP = N
RPW = NP // NS
R = 1000


def _tc1_body(f_ref, t_ref, wphi_ref, bphi_ref, whg_ref, phi_ref, xw_ref):
    phi = jnp.maximum(
        jnp.dot(f_ref[...], wphi_ref[...], preferred_element_type=jnp.float32)
        + bphi_ref[...], 0.0)
    phi_ref[...] = phi
    xw_ref[...] = jnp.dot(t_ref[...] * phi, whg_ref[...],
                          preferred_element_type=jnp.float32)


_tc1 = pl.pallas_call(
    _tc1_body,
    grid=(N // R,),
    in_specs=[
        pl.BlockSpec((R, XD), lambda i: (i, 0)),
        pl.BlockSpec((R, 1), lambda i: (i, 0)),
        pl.BlockSpec((XD, HD), lambda i: (0, 0)),
        pl.BlockSpec((1, HD), lambda i: (0, 0)),
        pl.BlockSpec((HD, GD), lambda i: (0, 0)),
    ],
    out_specs=[
        pl.BlockSpec((R, HD), lambda i: (i, 0)),
        pl.BlockSpec((R, GD), lambda i: (i, 0)),
    ],
    out_shape=[
        jax.ShapeDtypeStruct((N, HD), jnp.float32),
        jax.ShapeDtypeStruct((N, GD), jnp.float32),
    ],
)


def _tc2_body(pe_ref, pb_ref, ef_ref):
    s = pe_ref[0] + pe_ref[1]
    b = pb_ref[0] + pb_ref[1]
    binv = jnp.where(b > 0, 1.0 / jnp.maximum(b, 1.0), 0.0)
    ef_ref[...] = binv * s


_tc2 = pl.pallas_call(
    _tc2_body,
    grid=(N // R,),
    in_specs=[
        pl.BlockSpec((NC, R, GD), lambda i: (0, i, 0)),
        pl.BlockSpec((NC, R, 1), lambda i: (0, i, 0)),
    ],
    out_specs=pl.BlockSpec((R, GD), lambda i: (i, 0)),
    out_shape=jax.ShapeDtypeStruct((N, GD), jnp.float32),
)


def _tc3_body(pn_ref, pd_ref, phi_ref, bias_ref,
              w00a_ref, w00b_ref, b00_ref,
              w10a_ref, w10b_ref, b10_ref,
              w01_ref, b01_ref, w11_ref, b11_ref,
              y1_ref, y0_ref):
    q = pn_ref[0] + pn_ref[1]
    d = pd_ref[0] + pd_ref[1]
    dinv = jnp.where(d > 0, 1.0 / jnp.maximum(d, 1.0), 0.0)
    rep = dinv * q + bias_ref[...]
    phi = phi_ref[...]
    y00 = jnp.maximum(
        jnp.dot(phi, w00a_ref[...], preferred_element_type=jnp.float32)
        + jnp.dot(rep, w00b_ref[...], preferred_element_type=jnp.float32)
        + b00_ref[...], 0.0)
    y10 = jnp.maximum(
        jnp.dot(phi, w10a_ref[...], preferred_element_type=jnp.float32)
        + jnp.dot(rep, w10b_ref[...], preferred_element_type=jnp.float32)
        + b10_ref[...], 0.0)
    y0_ref[...] = jnp.dot(y00, w01_ref[...],
                          preferred_element_type=jnp.float32) + b01_ref[...]
    y1_ref[...] = jnp.dot(y10, w11_ref[...],
                          preferred_element_type=jnp.float32) + b11_ref[...]


_tc3 = pl.pallas_call(
    _tc3_body,
    grid=(N // R,),
    in_specs=[
        pl.BlockSpec((NC, R, GD), lambda i: (0, i, 0)),
        pl.BlockSpec((NC, R, 1), lambda i: (0, i, 0)),
        pl.BlockSpec((R, HD), lambda i: (i, 0)),
        pl.BlockSpec((1, GD), lambda i: (0, 0)),
        pl.BlockSpec((HD, PD), lambda i: (0, 0)),
        pl.BlockSpec((GD, PD), lambda i: (0, 0)),
        pl.BlockSpec((1, PD), lambda i: (0, 0)),
        pl.BlockSpec((HD, PD), lambda i: (0, 0)),
        pl.BlockSpec((GD, PD), lambda i: (0, 0)),
        pl.BlockSpec((1, PD), lambda i: (0, 0)),
        pl.BlockSpec((PD, 1), lambda i: (0, 0)),
        pl.BlockSpec((1, 1), lambda i: (0, 0)),
        pl.BlockSpec((PD, 1), lambda i: (0, 0)),
        pl.BlockSpec((1, 1), lambda i: (0, 0)),
    ],
    out_specs=[
        pl.BlockSpec((R, 1), lambda i: (i, 0)),
        pl.BlockSpec((R, 1), lambda i: (i, 0)),
    ],
    out_shape=[
        jax.ShapeDtypeStruct((N, 1), jnp.float32),
        jax.ShapeDtypeStruct((N, 1), jnp.float32),
    ],
)


def _build_sc_pass(mesh, with_counts):
    out_type = [jax.ShapeDtypeStruct((NC, NP, GD), jnp.float32)]
    scratch = [
        pltpu.VMEM_SHARED((NP, GD), jnp.float32),
        pltpu.VMEM((2, NBJ, K), jnp.int32),
        pltpu.VMEM((2, NBJ, K), jnp.int32),
        pltpu.VMEM((NBJ, K, GD), jnp.float32),
    ]
    if with_counts:
        out_type += [jax.ShapeDtypeStruct((NC, NP), jnp.float32),
                     jax.ShapeDtypeStruct((NC, NP), jnp.float32)]
        scratch += [pltpu.VMEM_SHARED((NP,), jnp.float32),
                    pltpu.VMEM_SHARED((NP,), jnp.float32),
                    pltpu.VMEM((K,), jnp.float32)]
    scratch += [pltpu.SemaphoreType.DMA] * (2 * NBJ + 1)

    @functools.partial(pl.kernel, mesh=mesh, out_type=out_type,
                       scratch_types=scratch)
    def sc_pass(*refs):
        if with_counts:
            (table, gidx, sidx, zrows, zcnt, ones_hbm,
             acc_out, pb_out, pd_out,
             acc, gring, sring, rows, bcnt, dcnt, ones, *sems) = refs
        else:
            (table, gidx, sidx, zrows,
             acc_out, acc, gring, sring, rows, *sems) = refs
        gsem = sems[:NBJ]
        ssem = sems[NBJ:2 * NBJ]
        isem = sems[2 * NBJ]
        cid = lax.axis_index("c")
        sid = lax.axis_index("s")

        pltpu.sync_copy(zrows, acc.at[pl.ds(sid * RPW, RPW)])
        if with_counts:
            @pl.when(sid == 0)
            def _zero_counts():
                pltpu.sync_copy(zcnt, bcnt)
                pltpu.sync_copy(zcnt, dcnt)
            pltpu.sync_copy(ones_hbm, ones)
        pltpu.sync_copy(gidx.at[cid, sid, pl.ds(0, NBJ)], gring.at[0])
        pltpu.sync_copy(sidx.at[cid, sid, pl.ds(0, NBJ)], sring.at[0])
        plsc.subcore_barrier()

        def outer(t, carry):
            p = lax.rem(t, 2)
            q = 1 - p
            tn = jnp.minimum(t + 1, NBURST - 1)
            dg = [pltpu.async_copy(table.at[gring.at[p, j]], rows.at[j],
                                   gsem[j])
                  for j in range(NBJ)]
            di = [pltpu.async_copy(gidx.at[cid, sid, pl.ds(tn * NBJ, NBJ)],
                                   gring.at[q], isem),
                  pltpu.async_copy(sidx.at[cid, sid, pl.ds(tn * NBJ, NBJ)],
                                   sring.at[q], isem)]
            ds = []
            for j in range(NBJ):
                dg[j].wait()
                ds.append(pltpu.async_copy(rows.at[j], acc.at[sring.at[p, j]],
                                           ssem[j], add=True))
                if with_counts:
                    ds.append(pltpu.async_copy(ones, bcnt.at[sring.at[p, j]],
                                               ssem[j], add=True))
                    ds.append(pltpu.async_copy(ones, dcnt.at[gring.at[p, j]],
                                               ssem[j], add=True))
            for d in ds + di:
                d.wait()
            return carry

        lax.fori_loop(0, NBURST, outer, 0)
        plsc.subcore_barrier()
        pltpu.sync_copy(acc.at[pl.ds(sid * RPW, RPW)],
                        acc_out.at[cid, pl.ds(sid * RPW, RPW)])
        if with_counts:
            @pl.when(sid == 0)
            def _flush_counts():
                pltpu.sync_copy(bcnt, pb_out.at[cid])
                pltpu.sync_copy(dcnt, pd_out.at[cid])

    return sc_pass


@functools.lru_cache(maxsize=1)
def _sc_kernels():
    mesh = plsc.VectorSubcoreMesh(core_axis_name="c", subcore_axis_name="s")
    return _build_sc_pass(mesh, True), _build_sc_pass(mesh, False)


def kernel(features, treatments, hyperedge_index, W_phi, b_phi, W_hg, bias_hg,
           W_t00, b_t00, W_t10, b_t10, W_t01, b_t01, W_t11, b_t11):
    phi_x, xw = _tc1(features, treatments.reshape(N, 1), W_phi,
                     b_phi.reshape(1, HD), W_hg)

    node_idx = hyperedge_index[0]
    edge_idx = hyperedge_index[1]
    pad = EP - E
    pad0 = jnp.zeros((pad,), jnp.int32)
    padd = jnp.full((pad,), DUMP, jnp.int32)
    n_g = jnp.concatenate([node_idx, pad0]).reshape(NC, NS, GI, K)
    n_s = jnp.concatenate([node_idx, padd]).reshape(NC, NS, GI, K)
    e_g = jnp.concatenate([edge_idx, pad0]).reshape(NC, NS, GI, K)
    e_s = jnp.concatenate([edge_idx, padd]).reshape(NC, NS, GI, K)

    zrows = jnp.zeros((RPW, GD), jnp.float32)
    zcnt = jnp.zeros((NP,), jnp.float32)
    onearr = jnp.ones((K,), jnp.float32)

    sc_pass_a, sc_pass_b = _sc_kernels()
    pe, pb, pd = sc_pass_a(xw, n_g, e_s, zrows, zcnt, onearr)
    ef = _tc2(pe, pb.reshape(NC, NP, 1))
    [pn] = sc_pass_b(ef, e_g, n_s, zrows)
    y1, y0 = _tc3(pn, pd.reshape(NC, NP, 1), phi_x, bias_hg.reshape(1, GD),
                  W_t00[:HD], W_t00[HD:], b_t00.reshape(1, PD),
                  W_t10[:HD], W_t10[HD:], b_t10.reshape(1, PD),
                  W_t01, b_t01.reshape(1, 1),
                  W_t11, b_t11.reshape(1, 1))
    return y1.reshape(-1), y0.reshape(-1), phi_x
